# baseline (speedup 1.0000x reference)
import jax
import jax.numpy as jnp
from jax.experimental import pallas as pl
from jax.experimental.pallas import tpu as pltpu

N = 10000
D_IN = 128
D_OUT = 128
BM = 200
NBLK = N // BM
NBUF = 4


def _gcn_kernel(x_ref, adj_ref, w_ref, b_ref, out_ref, buf, support, sems):
    def block_copy(i, slot):
        return pltpu.make_async_copy(
            adj_ref.at[pl.ds(i * BM, BM), :], buf.at[slot], sems.at[slot]
        )

    for i in range(NBUF):
        block_copy(i, i).start()

    support[...] = jnp.dot(
        x_ref[...], w_ref[...], preferred_element_type=jnp.float32
    )

    for i in range(NBLK):
        slot = i % NBUF
        block_copy(i, slot).wait()
        out_ref[pl.ds(i * BM, BM), :] = (
            jnp.dot(buf[slot], support[...], preferred_element_type=jnp.float32)
            + b_ref[...]
        )
        nxt = i + NBUF
        if nxt < NBLK:
            block_copy(nxt, slot).start()


@jax.jit
def kernel(input, adj, weight, bias):
    bias2d = bias.reshape(1, D_OUT)
    return pl.pallas_call(
        _gcn_kernel,
        in_specs=[
            pl.BlockSpec(memory_space=pltpu.VMEM),
            pl.BlockSpec(memory_space=pl.ANY),
            pl.BlockSpec(memory_space=pltpu.VMEM),
            pl.BlockSpec(memory_space=pltpu.VMEM),
        ],
        out_specs=pl.BlockSpec(memory_space=pltpu.VMEM),
        out_shape=jax.ShapeDtypeStruct((N, D_OUT), jnp.float32),
        scratch_shapes=[
            pltpu.VMEM((NBUF, BM, N), jnp.float32),
            pltpu.VMEM((N, D_OUT), jnp.float32),
            pltpu.SemaphoreType.DMA((NBUF,)),
        ],
    )(input, adj, weight, bias2d)


# speedup vs baseline: 1.0255x; 1.0255x over previous
import jax
import jax.numpy as jnp
from jax.experimental import pallas as pl
from jax.experimental.pallas import tpu as pltpu

N = 10000
D_IN = 128
D_OUT = 128
BM = 400


def _gcn_kernel(x_ref, adj_ref, w_ref, b_ref, out_ref, support_ref):
    @pl.when(pl.program_id(0) == 0)
    def _compute_support():
        support_ref[...] = jnp.dot(
            x_ref[...], w_ref[...], preferred_element_type=jnp.float32
        )

    acc = jnp.dot(
        adj_ref[...], support_ref[...], preferred_element_type=jnp.float32
    )
    out_ref[...] = acc + b_ref[...]


@jax.jit
def kernel(input, adj, weight, bias):
    bias2d = bias.reshape(1, D_OUT)
    return pl.pallas_call(
        _gcn_kernel,
        grid=(N // BM,),
        in_specs=[
            pl.BlockSpec((N, D_IN), lambda i: (0, 0)),
            pl.BlockSpec((BM, N), lambda i: (i, 0)),
            pl.BlockSpec((D_IN, D_OUT), lambda i: (0, 0)),
            pl.BlockSpec((1, D_OUT), lambda i: (0, 0)),
        ],
        out_specs=pl.BlockSpec((BM, D_OUT), lambda i: (i, 0)),
        out_shape=jax.ShapeDtypeStruct((N, D_OUT), jnp.float32),
        scratch_shapes=[pltpu.VMEM((N, D_OUT), jnp.float32)],
    )(input, adj, weight, bias2d)
